# baseline (speedup 1.0000x reference)
import functools

import jax
import jax.numpy as jnp
from jax import lax
from jax.experimental import pallas as pl
from jax.experimental.pallas import tpu as pltpu
from jax.experimental.pallas import tpu_sc as plsc

VOCAB = 1000000
HID = 64
MAXPOS = 200
B = 4096
L = 200

NC = 2
NS = 16
NW = NC * NS
BB = B // NW
NBB = BB // 16
NQ = HID // 16


def _body(ids_hbm, vec_hbm, wtab_hbm, ptab_hbm, out_hbm,
          idx_all, g0, g1, g2, in0, in1, in2, o0, o1, o2, pos_v,
          gsem0, gsem1, gsem2, osem0, osem1, osem2, vsem):
  wid = lax.axis_index("s") * NC + lax.axis_index("c")
  b0 = wid * BB

  gidx = (g0, g1, g2)
  slab_in = (in0, in1, in2)
  slab_out = (o0, o1, o2)
  gsem = (gsem0, gsem1, gsem2)
  osem = (osem0, osem1, osem2)

  iota = lax.broadcasted_iota(jnp.int32, (16,), 0)
  brow = [iota + bb * 16 for bb in range(NBB)]

  def prep_and_issue_gather(s, b):
    for bb in range(NBB):
      v = idx_all[s, pl.ds(bb * 16, 16)]
      gidx[b][pl.ds(bb * 16, 16)] = lax.shift_right_logical(v, 1)
    for k in range(4):
      pltpu.async_copy(wtab_hbm.at[gidx[b].at[pl.ds(k * (BB // 4), BB // 4)]],
                       slab_in[b].at[pl.ds(k * (BB // 4), BB // 4)],
                       gsem[b])

  def wait_gather(b):
    pltpu.make_async_copy(wtab_hbm.at[pl.ds(0, BB)], slab_in[b],
                          gsem[b]).wait()

  def issue_out(s, b):
    pltpu.async_copy(slab_out[b], out_hbm.at[1 + s, :, pl.ds(b0, BB)],
                     osem[b])

  def wait_out(b):
    pltpu.make_async_copy(slab_out[b], out_hbm.at[0, :, pl.ds(b0, BB)],
                          osem[b]).wait()

  pltpu.sync_copy(ids_hbm.at[:, pl.ds(b0, BB)], idx_all)
  pltpu.sync_copy(ptab_hbm.at[pl.ds(0, L)], pos_v)

  pltpu.sync_copy(vec_hbm.at[:, pl.ds(b0, BB)], slab_out[0])
  pltpu.async_copy(slab_out[0], out_hbm.at[0, :, pl.ds(b0, BB)], vsem)
  pltpu.make_async_copy(slab_out[0], out_hbm.at[0, :, pl.ds(b0, BB)],
                        vsem).wait()

  prep_and_issue_gather(0, 0)
  prep_and_issue_gather(1, 1)

  @pl.loop(0, (L + 2) // 3)
  def _tri(gg):
    for r in range(3):
      s = gg * 3 + r

      @pl.when(s + 2 < L)
      def _():
        prep_and_issue_gather(s + 2, (r + 2) % 3)

      @pl.when(s < L)
      def _():
        wait_gather(r)

        @pl.when(s >= 3)
        def _():
          wait_out(r)

        svec = jnp.full((16,), s, jnp.int32)
        half = [
            lax.shift_left(
                lax.bitwise_and(idx_all[s, pl.ds(bb * 16, 16)], 1), 6)
            for bb in range(NBB)
        ]

        @plsc.parallel_loop(0, HID, unroll=4)
        def _h(h):
          hvec = jnp.full((16,), h, jnp.int32)
          p = plsc.load_gather(pos_v, [svec, hvec])
          for bb in range(NBB):
            y = plsc.load_gather(slab_in[r], [brow[bb], half[bb] + hvec]) + p
            slab_out[r][h, pl.ds(bb * 16, 16)] = y

        issue_out(s, r)

  wait_out(0)
  wait_out(1)
  wait_out(2)


def kernel(input_ids, vectors, word_table, pos_table):
  ids_t = input_ids.T
  vec_t = vectors.T
  wtab2 = word_table.reshape(VOCAB // 2, 2 * HID)
  pos_block = lax.slice_in_dim(pos_table, 1, MAXPOS + 1, axis=0)
  mesh = plsc.VectorSubcoreMesh(core_axis_name="c", subcore_axis_name="s",
                                num_cores=NC, num_subcores=NS)
  out_t = pl.kernel(
      _body,
      out_type=jax.ShapeDtypeStruct((MAXPOS + 1, HID, B), jnp.float32),
      mesh=mesh,
      compiler_params=pltpu.CompilerParams(needs_layout_passes=False),
      scratch_types=[
          pltpu.VMEM((L, BB), jnp.int32),
          pltpu.VMEM((BB,), jnp.int32),
          pltpu.VMEM((BB,), jnp.int32),
          pltpu.VMEM((BB,), jnp.int32),
          pltpu.VMEM((BB, 2 * HID), jnp.float32),
          pltpu.VMEM((BB, 2 * HID), jnp.float32),
          pltpu.VMEM((BB, 2 * HID), jnp.float32),
          pltpu.VMEM((HID, BB), jnp.float32),
          pltpu.VMEM((HID, BB), jnp.float32),
          pltpu.VMEM((HID, BB), jnp.float32),
          pltpu.VMEM((L, HID), jnp.float32),
          pltpu.SemaphoreType.DMA,
          pltpu.SemaphoreType.DMA,
          pltpu.SemaphoreType.DMA,
          pltpu.SemaphoreType.DMA,
          pltpu.SemaphoreType.DMA,
          pltpu.SemaphoreType.DMA,
          pltpu.SemaphoreType.DMA,
      ],
  )(ids_t, vec_t, wtab2, pos_block)
  return jnp.transpose(out_t, (2, 0, 1))


# speedup vs baseline: 1.0547x; 1.0547x over previous
import functools

import jax
import jax.numpy as jnp
from jax import lax
from jax.experimental import pallas as pl
from jax.experimental.pallas import tpu as pltpu
from jax.experimental.pallas import tpu_sc as plsc

VOCAB = 1000000
HID = 64
MAXPOS = 200
B = 4096
L = 200

NC = 2
NS = 16
NW = NC * NS
BB = B // NW
NBB = BB // 16
NQ = HID // 16


def _body(ids_hbm, vec_hbm, wtab_hbm, ptab_hbm, out_hbm,
          idx_all, in0, in1, in2, o0, o1, o2, pos_v,
          gsem0, gsem1, gsem2, osem0, osem1, osem2, vsem):
  wid = lax.axis_index("s") * NC + lax.axis_index("c")
  b0 = wid * BB

  slab_in = (in0, in1, in2)
  slab_out = (o0, o1, o2)
  gsem = (gsem0, gsem1, gsem2)
  osem = (osem0, osem1, osem2)

  iota = lax.broadcasted_iota(jnp.int32, (16,), 0)
  brow = [iota + bb * 16 for bb in range(NBB)]

  def prep_and_issue_gather(s, b):
    for k in range(4):
      pltpu.async_copy(
          wtab_hbm.at[idx_all.at[s, pl.ds(k * (BB // 4), BB // 4)]],
          slab_in[b].at[pl.ds(k * (BB // 4), BB // 4)],
          gsem[b])

  def wait_gather(b):
    pltpu.make_async_copy(wtab_hbm.at[pl.ds(0, BB)],
                          slab_in[b],
                          gsem[b]).wait()

  def issue_out(s, b):
    pltpu.async_copy(slab_out[b], out_hbm.at[1 + s, :, pl.ds(b0, BB)],
                     osem[b])

  def wait_out(b):
    pltpu.make_async_copy(slab_out[b], out_hbm.at[0, :, pl.ds(b0, BB)],
                          osem[b]).wait()

  pltpu.sync_copy(ids_hbm.at[:, pl.ds(b0, BB)], idx_all)
  pltpu.sync_copy(ptab_hbm.at[pl.ds(0, L)], pos_v)

  pltpu.sync_copy(vec_hbm.at[:, pl.ds(b0, BB)], slab_out[0])
  pltpu.async_copy(slab_out[0], out_hbm.at[0, :, pl.ds(b0, BB)], vsem)
  pltpu.make_async_copy(slab_out[0], out_hbm.at[0, :, pl.ds(b0, BB)],
                        vsem).wait()

  prep_and_issue_gather(0, 0)
  prep_and_issue_gather(1, 1)

  @pl.loop(0, (L + 2) // 3)
  def _tri(gg):
    for r in range(3):
      s = gg * 3 + r

      @pl.when(s + 2 < L)
      def _():
        prep_and_issue_gather(s + 2, (r + 2) % 3)

      @pl.when(s < L)
      def _():
        wait_gather(r)

        @pl.when(s >= 3)
        def _():
          wait_out(r)

        svec = jnp.full((16,), s, jnp.int32)

        @plsc.parallel_loop(0, HID, unroll=4)
        def _h(h):
          hvec = jnp.full((16,), h, jnp.int32)
          p = plsc.load_gather(pos_v, [svec, hvec])
          for bb in range(NBB):
            y = plsc.load_gather(slab_in[r], [brow[bb], hvec]) + p
            slab_out[r][h, pl.ds(bb * 16, 16)] = y

        issue_out(s, r)

  wait_out(0)
  wait_out(1)
  wait_out(2)


def kernel(input_ids, vectors, word_table, pos_table):
  ids_t = input_ids.T
  vec_t = vectors.T
  wtab2 = jnp.pad(word_table, ((0, 0), (0, 128 - HID)))
  pos_block = lax.slice_in_dim(pos_table, 1, MAXPOS + 1, axis=0)
  mesh = plsc.VectorSubcoreMesh(core_axis_name="c", subcore_axis_name="s",
                                num_cores=NC, num_subcores=NS)
  out_t = pl.kernel(
      _body,
      out_type=jax.ShapeDtypeStruct((MAXPOS + 1, HID, B), jnp.float32),
      mesh=mesh,
      compiler_params=pltpu.CompilerParams(needs_layout_passes=False),
      scratch_types=[
          pltpu.VMEM((L, BB), jnp.int32),
          pltpu.VMEM((BB, 128), jnp.float32),
          pltpu.VMEM((BB, 128), jnp.float32),
          pltpu.VMEM((BB, 128), jnp.float32),
          pltpu.VMEM((HID, BB), jnp.float32),
          pltpu.VMEM((HID, BB), jnp.float32),
          pltpu.VMEM((HID, BB), jnp.float32),
          pltpu.VMEM((L, HID), jnp.float32),
          pltpu.SemaphoreType.DMA,
          pltpu.SemaphoreType.DMA,
          pltpu.SemaphoreType.DMA,
          pltpu.SemaphoreType.DMA,
          pltpu.SemaphoreType.DMA,
          pltpu.SemaphoreType.DMA,
          pltpu.SemaphoreType.DMA,
      ],
  )(ids_t, vec_t, wtab2, pos_block)
  return jnp.transpose(out_t, (2, 0, 1))


# speedup vs baseline: 1.7050x; 1.6165x over previous
import functools

import jax
import jax.numpy as jnp
from jax import lax
from jax.experimental import pallas as pl
from jax.experimental.pallas import tpu as pltpu
from jax.experimental.pallas import tpu_sc as plsc

VOCAB = 1000000
HID = 64
MAXPOS = 200
B = 4096
L = 200

NC = 2
NS = 16
NW = NC * NS
BB = B // NW
NBB = BB // 16
NQ = HID // 16


def _body(ids_hbm, vec_hbm, wtab_hbm, ptab_hbm, out_hbm,
          idx_all, in0, in1, in2, o0, o1, o2, pos_v,
          gsem0, gsem1, gsem2, osem0, osem1, osem2, vsem):
  wid = lax.axis_index("s") * NC + lax.axis_index("c")
  b0 = wid * BB

  slab_in = (in0, in1, in2)
  slab_out = (o0, o1, o2)
  gsem = (gsem0, gsem1, gsem2)
  osem = (osem0, osem1, osem2)

  iota = lax.broadcasted_iota(jnp.int32, (16,), 0)
  brow = [iota + bb * 16 for bb in range(NBB)]

  def prep_and_issue_gather(s, b):
    for k in range(4):
      pltpu.async_copy(
          wtab_hbm.at[idx_all.at[s, pl.ds(k * (BB // 4), BB // 4)]],
          slab_in[b].at[pl.ds(k * (BB // 4), BB // 4)],
          gsem[b])

  def wait_gather(b):
    pltpu.make_async_copy(wtab_hbm.at[pl.ds(0, BB)],
                          slab_in[b],
                          gsem[b]).wait()

  def issue_out(s, b):
    pltpu.async_copy(slab_out[b], out_hbm.at[1 + s, :, pl.ds(b0, BB)],
                     osem[b])

  def wait_out(b):
    pltpu.make_async_copy(slab_out[b], out_hbm.at[0, :, pl.ds(b0, BB)],
                          osem[b]).wait()

  pltpu.sync_copy(ids_hbm.at[:, pl.ds(b0, BB)], idx_all)
  pltpu.sync_copy(ptab_hbm.at[pl.ds(0, L)], pos_v)

  pltpu.sync_copy(vec_hbm.at[:, pl.ds(b0, BB)], slab_out[0])
  pltpu.async_copy(slab_out[0], out_hbm.at[0, :, pl.ds(b0, BB)], vsem)
  pltpu.make_async_copy(slab_out[0], out_hbm.at[0, :, pl.ds(b0, BB)],
                        vsem).wait()

  prep_and_issue_gather(0, 0)
  prep_and_issue_gather(1, 1)

  @pl.loop(0, (L + 2) // 3)
  def _tri(gg):
    for r in range(3):
      s = gg * 3 + r

      @pl.when(s + 2 < L)
      def _():
        prep_and_issue_gather(s + 2, (r + 2) % 3)

      @pl.when(s < L)
      def _():
        wait_gather(r)

        @pl.when(s >= 3)
        def _():
          wait_out(r)

        svec = jnp.full((16,), s, jnp.int32)

        @plsc.parallel_loop(0, HID, unroll=4)
        def _t(t):
          tvec = jnp.full((16,), t, jnp.int32)
          colv = lax.bitwise_and(iota + tvec, 15) + lax.bitwise_and(tvec, 48)
          p = plsc.load_gather(pos_v, [svec, colv])
          for bb in range(NBB):
            y = plsc.load_gather(slab_in[r], [brow[bb], colv]) + p
            plsc.store_scatter(slab_out[r], [colv, brow[bb]], y)

        issue_out(s, r)

  wait_out(0)
  wait_out(1)
  wait_out(2)


def kernel(input_ids, vectors, word_table, pos_table):
  ids_t = input_ids.T
  vec_t = vectors.T
  wtab2 = jnp.pad(word_table, ((0, 0), (0, 128 - HID)))
  pos_block = lax.slice_in_dim(pos_table, 1, MAXPOS + 1, axis=0)
  mesh = plsc.VectorSubcoreMesh(core_axis_name="c", subcore_axis_name="s",
                                num_cores=NC, num_subcores=NS)
  out_t = pl.kernel(
      _body,
      out_type=jax.ShapeDtypeStruct((MAXPOS + 1, HID, B), jnp.float32),
      mesh=mesh,
      compiler_params=pltpu.CompilerParams(needs_layout_passes=False),
      scratch_types=[
          pltpu.VMEM((L, BB), jnp.int32),
          pltpu.VMEM((BB, 128), jnp.float32),
          pltpu.VMEM((BB, 128), jnp.float32),
          pltpu.VMEM((BB, 128), jnp.float32),
          pltpu.VMEM((HID, BB), jnp.float32),
          pltpu.VMEM((HID, BB), jnp.float32),
          pltpu.VMEM((HID, BB), jnp.float32),
          pltpu.VMEM((L, HID), jnp.float32),
          pltpu.SemaphoreType.DMA,
          pltpu.SemaphoreType.DMA,
          pltpu.SemaphoreType.DMA,
          pltpu.SemaphoreType.DMA,
          pltpu.SemaphoreType.DMA,
          pltpu.SemaphoreType.DMA,
          pltpu.SemaphoreType.DMA,
      ],
  )(ids_t, vec_t, wtab2, pos_block)
  return jnp.transpose(out_t, (2, 0, 1))


# speedup vs baseline: 1.7057x; 1.0004x over previous
import jax
import jax.numpy as jnp
from jax import lax
from jax.experimental import pallas as pl
from jax.experimental.pallas import tpu as pltpu
from jax.experimental.pallas import tpu_sc as plsc

VOCAB = 1000000
HID = 64
MAXPOS = 200
B = 4096
L = 200

NC = 2
NS = 16
NW = NC * NS
BB = B // NW
NBB = BB // 16
NQ = HID // 16


def _body(ids_hbm, vec_hbm, wtab_hbm, ptab_hbm, out_hbm,
          idx_all, in0, in1, in2, o0, o1, o2, pos_v,
          gsem0, gsem1, gsem2, osem0, osem1, osem2, vsem):
  wid = lax.axis_index("s") * NC + lax.axis_index("c")
  b0 = wid * BB

  slab_in = (in0, in1, in2)
  slab_out = (o0, o1, o2)
  gsem = (gsem0, gsem1, gsem2)
  osem = (osem0, osem1, osem2)

  iota = lax.broadcasted_iota(jnp.int32, (16,), 0)
  brow = [iota + bb * 16 for bb in range(NBB)]

  def prep_and_issue_gather(s, b):
    for k in range(4):
      pltpu.async_copy(
          wtab_hbm.at[idx_all.at[s, pl.ds(k * (BB // 4), BB // 4)]],
          slab_in[b].at[pl.ds(k * (BB // 4), BB // 4)],
          gsem[b])

  def wait_gather(b):
    pltpu.make_async_copy(wtab_hbm.at[pl.ds(0, BB)],
                          slab_in[b],
                          gsem[b]).wait()

  def issue_out(s, b):
    pltpu.async_copy(slab_out[b], out_hbm.at[1 + s, :, pl.ds(b0, BB)],
                     osem[b])

  def wait_out(b):
    pltpu.make_async_copy(slab_out[b], out_hbm.at[0, :, pl.ds(b0, BB)],
                          osem[b]).wait()

  pltpu.sync_copy(ids_hbm.at[:, pl.ds(b0, BB)], idx_all)
  pltpu.sync_copy(ptab_hbm.at[pl.ds(0, L)], pos_v)

  pltpu.sync_copy(vec_hbm.at[:, pl.ds(b0, BB)], slab_out[0])
  pltpu.async_copy(slab_out[0], out_hbm.at[0, :, pl.ds(b0, BB)], vsem)
  pltpu.make_async_copy(slab_out[0], out_hbm.at[0, :, pl.ds(b0, BB)],
                        vsem).wait()

  prep_and_issue_gather(0, 0)
  prep_and_issue_gather(1, 1)

  @pl.loop(0, (L + 2) // 3)
  def _tri(gg):
    for r in range(3):
      s = gg * 3 + r

      @pl.when(s + 2 < L)
      def _():
        prep_and_issue_gather(s + 2, (r + 2) % 3)

      @pl.when(s < L)
      def _():
        wait_gather(r)

        @pl.when(s >= 3)
        def _():
          wait_out(r)

        svec = jnp.full((16,), s, jnp.int32)

        @plsc.parallel_loop(0, HID, unroll=4)
        def _t(t):
          tvec = jnp.full((16,), t, jnp.int32)
          colv = lax.bitwise_and(iota + tvec, 15) + lax.bitwise_and(tvec, 48)
          p = plsc.load_gather(pos_v, [svec, colv])
          for bb in range(NBB):
            y = plsc.load_gather(slab_in[r], [brow[bb], colv]) + p
            plsc.store_scatter(slab_out[r], [colv, brow[bb]], y)

        issue_out(s, r)

  wait_out(0)
  wait_out(1)
  wait_out(2)


def kernel(input_ids, vectors, word_table, pos_table):
  ids_t = input_ids.T
  vec_t = vectors.T
  wtab2 = jnp.pad(word_table, ((0, 0), (0, 128 - HID)))
  pos_block = lax.slice_in_dim(pos_table, 1, MAXPOS + 1, axis=0)
  mesh = plsc.VectorSubcoreMesh(core_axis_name="c", subcore_axis_name="s",
                                num_cores=NC, num_subcores=NS)
  out_t = pl.kernel(
      _body,
      out_type=jax.ShapeDtypeStruct((MAXPOS + 1, HID, B), jnp.float32),
      mesh=mesh,
      compiler_params=pltpu.CompilerParams(needs_layout_passes=False),
      scratch_types=[
          pltpu.VMEM((L, BB), jnp.int32),
          pltpu.VMEM((BB, 128), jnp.float32),
          pltpu.VMEM((BB, 128), jnp.float32),
          pltpu.VMEM((BB, 128), jnp.float32),
          pltpu.VMEM((HID, BB), jnp.float32),
          pltpu.VMEM((HID, BB), jnp.float32),
          pltpu.VMEM((HID, BB), jnp.float32),
          pltpu.VMEM((L, HID), jnp.float32),
          pltpu.SemaphoreType.DMA,
          pltpu.SemaphoreType.DMA,
          pltpu.SemaphoreType.DMA,
          pltpu.SemaphoreType.DMA,
          pltpu.SemaphoreType.DMA,
          pltpu.SemaphoreType.DMA,
          pltpu.SemaphoreType.DMA,
      ],
  )(ids_t, vec_t, wtab2, pos_block)
  return jnp.transpose(out_t, (2, 0, 1))
